# baseline (speedup 1.0000x reference)
import functools

import jax
import jax.numpy as jnp
from jax import lax
from jax.experimental import pallas as pl
from jax.experimental.pallas import tpu as pltpu
from jax.experimental.pallas import tpu_sc as plsc

N = 10000
E = 320000
D = 128

NC = 2
NS = 16
NW = NC * NS
CH = 128
G = 4
NBLK = 20
NCHUNK = G * NBLK
EPT = NCHUNK * CH
E_PAD = NW * EPT
NPAD = 10240
RPT = NPAD // NS
DUMMY_DST = NPAD - 8


def _seg_body(x_hbm, srcs_hbm, dsts_hbm, zeros_hbm, out_hbm,
              sv0, sv1, dv0, dv1, r0, r1, agg_sh,
              i0, i1, g0, g1, s0, s1):
    c = lax.axis_index("c")
    s = lax.axis_index("s")
    tile = c * NS + s
    sv = (sv0, sv1)
    dv = (dv0, dv1)
    rows = (r0, r1)
    isems = (i0, i1)
    gsems = (g0, g1)
    ssems = (s0, s1)

    pltpu.sync_copy(zeros_hbm, r0)
    row0 = s * RPT
    for k in range(RPT // CH):
        pltpu.sync_copy(r0, agg_sh.at[pl.ds(row0 + k * CH, CH)])

    pltpu.sync_copy(srcs_hbm.at[tile, 0], sv0)
    pltpu.sync_copy(dsts_hbm.at[tile, 0], dv0)
    pltpu.async_copy(x_hbm.at[sv0.at[0]], r0, g0)
    pltpu.async_copy(x_hbm.at[sv0.at[1]], r1, g1)
    plsc.subcore_barrier()

    def do_block(ib, h):
        nxt = 1 - h

        @pl.when(ib + 1 < NBLK)
        def _():
            pltpu.async_copy(srcs_hbm.at[tile, ib + 1], sv[nxt], isems[nxt])
            pltpu.async_copy(dsts_hbm.at[tile, ib + 1], dv[nxt], isems[nxt])

        for cc in (0, 1):
            pltpu.make_async_copy(x_hbm.at[sv[h].at[cc]], rows[cc],
                                  gsems[cc]).wait()
            pltpu.async_copy(rows[cc], agg_sh.at[dv[h].at[cc]], ssems[cc],
                             add=True)

        for cc in (2, 3):
            b = cc - 2
            pltpu.make_async_copy(rows[b], agg_sh.at[dv[h].at[b]],
                                  ssems[b]).wait()
            pltpu.async_copy(x_hbm.at[sv[h].at[cc]], rows[b], gsems[b])
        for cc in (2, 3):
            b = cc - 2
            pltpu.make_async_copy(x_hbm.at[sv[h].at[cc]], rows[b],
                                  gsems[b]).wait()
            pltpu.async_copy(rows[b], agg_sh.at[dv[h].at[cc]], ssems[b],
                             add=True)

        @pl.when(ib + 1 < NBLK)
        def _():
            pltpu.make_async_copy(srcs_hbm.at[tile, ib + 1], sv[nxt],
                                  isems[nxt]).wait()
            pltpu.make_async_copy(dsts_hbm.at[tile, ib + 1], dv[nxt],
                                  isems[nxt]).wait()
            for b in (0, 1):
                pltpu.make_async_copy(rows[b], agg_sh.at[dv[h].at[b + 2]],
                                      ssems[b]).wait()
                pltpu.async_copy(x_hbm.at[sv[nxt].at[b]], rows[b], gsems[b])

    def group(g2, carry):
        do_block(2 * g2, 0)
        do_block(2 * g2 + 1, 1)
        return carry

    lax.fori_loop(0, NBLK // 2, group, 0)

    for b in (0, 1):
        pltpu.make_async_copy(rows[b], agg_sh.at[dv1.at[b + 2]],
                              ssems[b]).wait()
    plsc.subcore_barrier()

    pltpu.sync_copy(agg_sh.at[pl.ds(row0, RPT)],
                    out_hbm.at[c, pl.ds(row0, RPT)])


_segsum = functools.partial(
    pl.kernel,
    mesh=plsc.VectorSubcoreMesh(core_axis_name="c", subcore_axis_name="s"),
    out_type=jax.ShapeDtypeStruct((NC, NPAD, D), jnp.float32),
    scratch_types=[
        pltpu.VMEM((G, CH), jnp.int32),
        pltpu.VMEM((G, CH), jnp.int32),
        pltpu.VMEM((G, CH), jnp.int32),
        pltpu.VMEM((G, CH), jnp.int32),
        pltpu.VMEM((CH, D), jnp.float32),
        pltpu.VMEM((CH, D), jnp.float32),
        pltpu.VMEM_SHARED((NPAD, D), jnp.float32),
        pltpu.SemaphoreType.DMA,
        pltpu.SemaphoreType.DMA,
        pltpu.SemaphoreType.DMA,
        pltpu.SemaphoreType.DMA,
        pltpu.SemaphoreType.DMA,
        pltpu.SemaphoreType.DMA,
    ],
)(_seg_body)


BM = 1000


def _mlp_body(x_ref, p_ref, wa_ref, ba_ref, wb_ref, bb_ref, o_ref):
    t = x_ref[...] + p_ref[0] + p_ref[1]
    u = jnp.maximum(
        jnp.dot(t, wa_ref[...], preferred_element_type=jnp.float32)
        + ba_ref[...], 0.0)
    v = jnp.dot(u, wb_ref[...], preferred_element_type=jnp.float32) \
        + bb_ref[...]
    o_ref[...] = jnp.maximum(v, 0.0)


def _gin_dense(x, p, wa, ba, wb, bb):
    return pl.pallas_call(
        _mlp_body,
        grid=(N // BM,),
        in_specs=[
            pl.BlockSpec((BM, D), lambda i: (i, 0)),
            pl.BlockSpec((2, BM, D), lambda i: (0, i, 0)),
            pl.BlockSpec((D, D), lambda i: (0, 0)),
            pl.BlockSpec((1, D), lambda i: (0, 0)),
            pl.BlockSpec((D, D), lambda i: (0, 0)),
            pl.BlockSpec((1, D), lambda i: (0, 0)),
        ],
        out_specs=pl.BlockSpec((BM, D), lambda i: (i, 0)),
        out_shape=jax.ShapeDtypeStruct((N, D), jnp.float32),
    )(x, p, wa, ba, wb, bb)


@jax.jit
def kernel(x, edge_index, W0a, b0a, W0b, b0b, W1a, b1a, W1b, b1b):
    pad = E_PAD - E
    src = jnp.concatenate([edge_index[0],
                           jnp.zeros((pad,), jnp.int32)])
    dst = jnp.concatenate([edge_index[1],
                           jnp.full((pad,), DUMMY_DST, jnp.int32)])
    src = src.reshape(NW, NBLK, G, CH)
    dst = dst.reshape(NW, NBLK, G, CH)
    zeros = jnp.zeros((CH, D), jnp.float32)

    p = _segsum(x, src, dst, zeros)
    h = _gin_dense(x, p, W0a, b0a.reshape(1, D), W0b, b0b.reshape(1, D))
    p2 = _segsum(h, src, dst, zeros)
    out = _gin_dense(h, p2, W1a, b1a.reshape(1, D), W1b, b1b.reshape(1, D))
    return out
